# Initial kernel scaffold; baseline (speedup 1.0000x reference)
import jax
import jax.numpy as jnp
from jax.experimental import pallas as pl


def kernel(out, target, centers, batch_size, device):
    raise NotImplementedError("write your pallas kernel here")


# speedup vs baseline: 4.6178x; 4.6178x over previous
import functools

import jax
import jax.numpy as jnp
from jax import lax
from jax.experimental import pallas as pl
from jax.experimental.pallas import tpu as pltpu
from jax.experimental.pallas import tpu_sc as plsc

_DELTA_A = 0.1
_DELTA_R = 1.0
_ALPHA = 1.0
_BETA = 1.0
_GAMMA = 0.001


def _sc_gather(out_flat, tgt_flat, emb_idx, lab_idx):
    bk, d = emb_idx.shape
    info = plsc.get_sparse_core_info()
    nw = info.num_cores * info.num_subcores
    pairs = bk // nw

    @functools.partial(
        pl.kernel,
        mesh=plsc.VectorSubcoreMesh(core_axis_name="c", subcore_axis_name="s"),
        out_type=[
            jax.ShapeDtypeStruct((bk, d), jnp.float32),
            jax.ShapeDtypeStruct((bk,), jnp.int32),
        ],
        scratch_types=[
            pltpu.VMEM((pairs, d), jnp.int32),
            pltpu.VMEM((pairs, d), jnp.float32),
            pltpu.VMEM((bk,), jnp.int32),
            pltpu.VMEM((bk,), jnp.int32),
            pltpu.SemaphoreType.DMA,
        ],
    )
    def gather_kernel(out_hbm, tgt_hbm, eidx_hbm, lidx_hbm, c_out, lab_out,
                      eidx_v, vals_v, lidx_v, lvals_v, sem):
        wid = lax.axis_index("s") * info.num_cores + lax.axis_index("c")
        base = wid * pairs
        pltpu.sync_copy(eidx_hbm.at[pl.ds(base, pairs)], eidx_v)
        for k in range(pairs):
            pltpu.async_copy(out_hbm.at[eidx_v.at[k]], vals_v.at[k], sem).wait()
        pltpu.sync_copy(vals_v, c_out.at[pl.ds(base, pairs)])

        @pl.when(wid == 0)
        def _():
            pltpu.sync_copy(lidx_hbm, lidx_v)
            pltpu.async_copy(tgt_hbm.at[lidx_v], lvals_v, sem).wait()
            pltpu.sync_copy(lvals_v, lab_out)

    return gather_kernel(out_flat, tgt_flat, emb_idx, lab_idx)


def _tc_body(out_ref, tgt_ref, c_ref, lab_ref, res_ref, attr_ref, cnt_ref):
    j = pl.program_id(1)
    nb = pl.num_programs(1)
    o = out_ref[0]
    t = tgt_ref[0]
    c = c_ref[0]
    lab = lab_ref[0]
    k_centers = c.shape[0]

    @pl.when(j == 0)
    def _():
        attr_ref[...] = jnp.zeros_like(attr_ref)
        cnt_ref[...] = jnp.zeros_like(cnt_ref)

    g = lax.dot_general(c, o, (((1,), (0,)), ((), ())),
                        preferred_element_type=jnp.float32,
                        precision=lax.Precision.HIGHEST)
    pn2 = jnp.sum(o * o, axis=0, keepdims=True)
    cn2 = jnp.sum(c * c, axis=1, keepdims=True)
    sq = jnp.maximum(cn2 + pn2 - 2.0 * g, 0.0)
    hinged = jnp.maximum(jnp.sqrt(sq) - _DELTA_A, 0.0)
    m = (t == lab).astype(jnp.float32)
    attr_ref[:, :1] += jnp.sum(hinged * m, axis=1, keepdims=True)
    cnt_ref[:, :1] += jnp.sum(m, axis=1, keepdims=True)

    @pl.when(j == nb - 1)
    def _():
        n = cnt_ref[:, :1]
        denom = jnp.where(n > 1.0, n - 1.0, jnp.maximum(n, 1.0))
        a_i = jnp.sum(attr_ref[:, :1] / denom)
        r_i = jnp.float32(0.0)
        for jj in range(k_centers):
            dvec = c - lax.slice(c, (jj, 0), (jj + 1, c.shape[1]))
            sqd = jnp.sum(dvec * dvec, axis=1, keepdims=True)
            r_i += jnp.sum(jnp.maximum(_DELTA_R - jnp.sqrt(sqd), 0.0)) - _DELTA_R
        g_i = jnp.sum(jnp.sqrt(cn2))
        lanes = lax.broadcasted_iota(jnp.int32, (1, 128), 1)
        vec = jnp.where(lanes == 0, a_i,
                        jnp.where(lanes == 1, r_i,
                                  jnp.where(lanes == 2, g_i, 0.0)))
        res_ref[0] = vec


def _tc_main(out_r, tgt_r, c_r, lab_r, bn):
    b, d, hw = out_r.shape
    k = c_r.shape[1]
    nb = hw // bn
    return pl.pallas_call(
        _tc_body,
        grid=(b, nb),
        in_specs=[
            pl.BlockSpec((1, d, bn), lambda i, j: (i, 0, j)),
            pl.BlockSpec((1, 1, bn), lambda i, j: (i, 0, j)),
            pl.BlockSpec((1, k, d), lambda i, j: (i, 0, 0)),
            pl.BlockSpec((1, k, 1), lambda i, j: (i, 0, 0)),
        ],
        out_specs=pl.BlockSpec((1, 1, 128), lambda i, j: (i, 0, 0)),
        out_shape=jax.ShapeDtypeStruct((b, 1, 128), jnp.float32),
        scratch_shapes=[
            pltpu.VMEM((k, 128), jnp.float32),
            pltpu.VMEM((k, 128), jnp.float32),
        ],
        compiler_params=pltpu.CompilerParams(
            dimension_semantics=("arbitrary", "arbitrary"),
        ),
    )(out_r, tgt_r, c_r, lab_r)


def _assemble(res, b, k):
    a = res[:, 0, 0]
    r = res[:, 0, 1]
    g = res[:, 0, 2]
    att = jnp.float32(0.0)
    rep = jnp.float32(0.0)
    reg = jnp.float32(0.0)
    for i in range(b):
        att = (att + a[i]) / k
        rep = (rep + r[i]) / (k * (k - 1))
        reg = (reg + g[i]) / k
    loss = _ALPHA * att + _BETA * rep + _GAMMA * reg
    return (loss, att, rep)


def kernel(out, target, centers, batch_size, device):
    b, d, h, w = out.shape
    k = centers.shape[1]
    hw = h * w

    centers = centers.astype(jnp.int32)
    target = target.astype(jnp.int32)
    p = centers[..., 0] * w + centers[..., 1]
    bidx = jnp.arange(b, dtype=jnp.int32)[:, None]
    lab_idx = (bidx * hw + p).reshape(-1)
    d_ar = jnp.arange(d, dtype=jnp.int32)
    emb_idx = ((bidx[:, :, None] * d + d_ar[None, None, :]) * hw
               + p[:, :, None]).reshape(b * k, d)

    c_flat, lab_flat = _sc_gather(out.reshape(-1), target.reshape(-1),
                                  emb_idx, lab_idx)
    c_r = c_flat.reshape(b, k, d)
    lab_r = lab_flat.reshape(b, k, 1)

    res = _tc_main(out.reshape(b, d, hw), target.reshape(b, 1, hw),
                   c_r, lab_r, bn=3584)
    return _assemble(res, b, k)


# speedup vs baseline: 4.8189x; 1.0435x over previous
import functools

import jax
import jax.numpy as jnp
from jax import lax
from jax.experimental import pallas as pl
from jax.experimental.pallas import tpu as pltpu
from jax.experimental.pallas import tpu_sc as plsc

_DELTA_A = 0.1
_DELTA_R = 1.0
_ALPHA = 1.0
_BETA = 1.0
_GAMMA = 0.001


def _sc_gather(out_flat, tgt_flat, emb_idx, lab_idx):
    bk, d = emb_idx.shape
    info = plsc.get_sparse_core_info()
    nw = info.num_cores * info.num_subcores
    pairs = bk // nw

    @functools.partial(
        pl.kernel,
        mesh=plsc.VectorSubcoreMesh(core_axis_name="c", subcore_axis_name="s"),
        out_type=[
            jax.ShapeDtypeStruct((bk, d), jnp.float32),
            jax.ShapeDtypeStruct((bk,), jnp.int32),
        ],
        scratch_types=[
            pltpu.VMEM((pairs, d), jnp.int32),
            pltpu.VMEM((pairs, d), jnp.float32),
            pltpu.VMEM((bk,), jnp.int32),
            pltpu.VMEM((bk,), jnp.int32),
            pltpu.SemaphoreType.DMA,
        ],
    )
    def gather_kernel(out_hbm, tgt_hbm, eidx_hbm, lidx_hbm, c_out, lab_out,
                      eidx_v, vals_v, lidx_v, lvals_v, sem):
        wid = lax.axis_index("s") * info.num_cores + lax.axis_index("c")
        base = wid * pairs
        pltpu.sync_copy(eidx_hbm.at[pl.ds(base, pairs)], eidx_v)
        for k in range(pairs):
            pltpu.async_copy(out_hbm.at[eidx_v.at[k]], vals_v.at[k], sem).wait()
        pltpu.sync_copy(vals_v, c_out.at[pl.ds(base, pairs)])

        @pl.when(wid == 0)
        def _():
            pltpu.sync_copy(lidx_hbm, lidx_v)
            pltpu.async_copy(tgt_hbm.at[lidx_v], lvals_v, sem).wait()
            pltpu.sync_copy(lvals_v, lab_out)

    return gather_kernel(out_flat, tgt_flat, emb_idx, lab_idx)


def _tc_body(out_ref, tgt_ref, c_ref, lab_ref, res_ref, attr_ref, cnt_ref):
    j = pl.program_id(1)
    nb = pl.num_programs(1)
    o = out_ref[0]
    t = tgt_ref[0]
    c = c_ref[0]
    lab = lab_ref[0]
    k_centers = c.shape[0]

    @pl.when(j == 0)
    def _():
        attr_ref[...] = jnp.zeros_like(attr_ref)
        cnt_ref[...] = jnp.zeros_like(cnt_ref)

    g = lax.dot_general(c, o, (((1,), (0,)), ((), ())),
                        preferred_element_type=jnp.float32,
                        precision=lax.Precision.DEFAULT)
    pn2 = jnp.sum(o * o, axis=0, keepdims=True)
    cn2 = jnp.sum(c * c, axis=1, keepdims=True)
    sq = jnp.maximum(cn2 + pn2 - 2.0 * g, 0.0)
    hinged = jnp.maximum(jnp.sqrt(sq) - _DELTA_A, 0.0)
    m = (t == lab).astype(jnp.float32)
    attr_ref[:, :1] += jnp.sum(hinged * m, axis=1, keepdims=True)
    cnt_ref[:, :1] += jnp.sum(m, axis=1, keepdims=True)

    @pl.when(j == nb - 1)
    def _():
        n = cnt_ref[:, :1]
        denom = jnp.where(n > 1.0, n - 1.0, jnp.maximum(n, 1.0))
        a_i = jnp.sum(attr_ref[:, :1] / denom)
        r_i = jnp.float32(0.0)
        for jj in range(k_centers):
            dvec = c - lax.slice(c, (jj, 0), (jj + 1, c.shape[1]))
            sqd = jnp.sum(dvec * dvec, axis=1, keepdims=True)
            r_i += jnp.sum(jnp.maximum(_DELTA_R - jnp.sqrt(sqd), 0.0)) - _DELTA_R
        g_i = jnp.sum(jnp.sqrt(cn2))
        lanes = lax.broadcasted_iota(jnp.int32, (1, 128), 1)
        vec = jnp.where(lanes == 0, a_i,
                        jnp.where(lanes == 1, r_i,
                                  jnp.where(lanes == 2, g_i, 0.0)))
        res_ref[0] = vec


def _tc_main(out_r, tgt_r, c_r, lab_r, bn):
    b, d, hw = out_r.shape
    k = c_r.shape[1]
    nb = hw // bn
    return pl.pallas_call(
        _tc_body,
        grid=(b, nb),
        in_specs=[
            pl.BlockSpec((1, d, bn), lambda i, j: (i, 0, j)),
            pl.BlockSpec((1, 1, bn), lambda i, j: (i, 0, j)),
            pl.BlockSpec((1, k, d), lambda i, j: (i, 0, 0)),
            pl.BlockSpec((1, k, 1), lambda i, j: (i, 0, 0)),
        ],
        out_specs=pl.BlockSpec((1, 1, 128), lambda i, j: (i, 0, 0)),
        out_shape=jax.ShapeDtypeStruct((b, 1, 128), jnp.float32),
        scratch_shapes=[
            pltpu.VMEM((k, 128), jnp.float32),
            pltpu.VMEM((k, 128), jnp.float32),
        ],
        compiler_params=pltpu.CompilerParams(
            dimension_semantics=("arbitrary", "arbitrary"),
        ),
    )(out_r, tgt_r, c_r, lab_r)


def _assemble(res, b, k):
    a = res[:, 0, 0]
    r = res[:, 0, 1]
    g = res[:, 0, 2]
    att = jnp.float32(0.0)
    rep = jnp.float32(0.0)
    reg = jnp.float32(0.0)
    for i in range(b):
        att = (att + a[i]) / k
        rep = (rep + r[i]) / (k * (k - 1))
        reg = (reg + g[i]) / k
    loss = _ALPHA * att + _BETA * rep + _GAMMA * reg
    return (loss, att, rep)


def kernel(out, target, centers, batch_size, device):
    b, d, h, w = out.shape
    k = centers.shape[1]
    hw = h * w

    centers = centers.astype(jnp.int32)
    target = target.astype(jnp.int32)
    p = centers[..., 0] * w + centers[..., 1]
    bidx = jnp.arange(b, dtype=jnp.int32)[:, None]
    lab_idx = (bidx * hw + p).reshape(-1)
    d_ar = jnp.arange(d, dtype=jnp.int32)
    emb_idx = ((bidx[:, :, None] * d + d_ar[None, None, :]) * hw
               + p[:, :, None]).reshape(b * k, d)

    c_flat, lab_flat = _sc_gather(out.reshape(-1), target.reshape(-1),
                                  emb_idx, lab_idx)
    c_r = c_flat.reshape(b, k, d)
    lab_r = lab_flat.reshape(b, k, 1)

    res = _tc_main(out.reshape(b, d, hw), target.reshape(b, 1, hw),
                   c_r, lab_r, bn=3584)
    return _assemble(res, b, k)


# speedup vs baseline: 6.7793x; 1.4068x over previous
import functools

import jax
import jax.numpy as jnp
from jax import lax
from jax.experimental import pallas as pl
from jax.experimental.pallas import tpu as pltpu
from jax.experimental.pallas import tpu_sc as plsc

_DELTA_A = 0.1
_DELTA_R = 1.0
_ALPHA = 1.0
_BETA = 1.0
_GAMMA = 0.001


def _sc_gather(out_flat, tgt_flat, emb_idx, lab_idx):
    bk, d = emb_idx.shape
    info = plsc.get_sparse_core_info()
    nw = info.num_cores * info.num_subcores
    pairs = bk // nw

    @functools.partial(
        pl.kernel,
        mesh=plsc.VectorSubcoreMesh(core_axis_name="c", subcore_axis_name="s"),
        out_type=[
            jax.ShapeDtypeStruct((bk, d), jnp.float32),
            jax.ShapeDtypeStruct((bk,), jnp.int32),
        ],
        scratch_types=[
            pltpu.VMEM((pairs, d), jnp.int32),
            pltpu.VMEM((pairs, d), jnp.float32),
            pltpu.VMEM((bk,), jnp.int32),
            pltpu.VMEM((bk,), jnp.int32),
            pltpu.SemaphoreType.DMA,
        ],
    )
    def gather_kernel(out_hbm, tgt_hbm, eidx_hbm, lidx_hbm, c_out, lab_out,
                      eidx_v, vals_v, lidx_v, lvals_v, sem):
        wid = lax.axis_index("s") * info.num_cores + lax.axis_index("c")
        base = wid * pairs
        pltpu.sync_copy(eidx_hbm.at[pl.ds(base, pairs)], eidx_v)
        for k in range(pairs):
            pltpu.async_copy(out_hbm.at[eidx_v.at[k]], vals_v.at[k], sem).wait()
        pltpu.sync_copy(vals_v, c_out.at[pl.ds(base, pairs)])

        @pl.when(wid == 0)
        def _():
            pltpu.sync_copy(lidx_hbm, lidx_v)
            pltpu.async_copy(tgt_hbm.at[lidx_v], lvals_v, sem).wait()
            pltpu.sync_copy(lvals_v, lab_out)

    return gather_kernel(out_flat, tgt_flat, emb_idx, lab_idx)


def _tc_gather_body(blk_ref, off_ref, out_ref, tgt_ref, c_ref, lab_ref):
    g = pl.program_id(0)
    po = off_ref[g]
    o = out_ref[0]
    t = tgt_ref[0]
    lanes = lax.broadcasted_iota(jnp.int32, (1, 128), 1)
    msk = (lanes == po).astype(jnp.float32)
    c_ref[0] = jnp.sum(o * msk, axis=1, keepdims=True)
    labv = jnp.sum(t * msk.astype(jnp.int32), axis=1, keepdims=True)
    lab_ref[0] = jnp.broadcast_to(labv, (1, 128))


def _tc_gather(out_r, tgt_r, blk, off):
    b, d, hw = out_r.shape
    bk = blk.shape[0]
    k = bk // b
    grid_spec = pltpu.PrefetchScalarGridSpec(
        num_scalar_prefetch=2,
        grid=(bk,),
        in_specs=[
            pl.BlockSpec((1, d, 128), lambda g, blk, off: (g // k, 0, blk[g])),
            pl.BlockSpec((1, 1, 128), lambda g, blk, off: (g // k, 0, blk[g])),
        ],
        out_specs=[
            pl.BlockSpec((1, d, 1), lambda g, blk, off: (g, 0, 0)),
            pl.BlockSpec((1, 1, 128), lambda g, blk, off: (g, 0, 0)),
        ],
    )
    c, lab = pl.pallas_call(
        _tc_gather_body,
        grid_spec=grid_spec,
        out_shape=[
            jax.ShapeDtypeStruct((bk, d, 1), jnp.float32),
            jax.ShapeDtypeStruct((bk, 1, 128), jnp.int32),
        ],
    )(blk, off, out_r, tgt_r)
    return c, lab


def _tc_body(out_ref, tgt_ref, c_ref, lab_ref, res_ref, attr_ref, cnt_ref):
    j = pl.program_id(1)
    nb = pl.num_programs(1)
    o = out_ref[0]
    t = tgt_ref[0]
    c = c_ref[0]
    lab = lab_ref[0]
    k_centers = c.shape[0]

    @pl.when(j == 0)
    def _():
        attr_ref[...] = jnp.zeros_like(attr_ref)
        cnt_ref[...] = jnp.zeros_like(cnt_ref)

    g = lax.dot_general(c, o, (((1,), (0,)), ((), ())),
                        preferred_element_type=jnp.float32,
                        precision=lax.Precision.DEFAULT)
    pn2 = jnp.sum(o * o, axis=0, keepdims=True)
    cn2 = jnp.sum(c * c, axis=1, keepdims=True)
    sq = jnp.maximum(cn2 + pn2 - 2.0 * g, 0.0)
    hinged = jnp.maximum(jnp.sqrt(sq) - _DELTA_A, 0.0)
    m = (t == lab).astype(jnp.float32)
    attr_ref[:, :1] += jnp.sum(hinged * m, axis=1, keepdims=True)
    cnt_ref[:, :1] += jnp.sum(m, axis=1, keepdims=True)

    @pl.when(j == nb - 1)
    def _():
        n = cnt_ref[:, :1]
        denom = jnp.where(n > 1.0, n - 1.0, jnp.maximum(n, 1.0))
        a_i = jnp.sum(attr_ref[:, :1] / denom)
        r_i = jnp.float32(0.0)
        for jj in range(k_centers):
            dvec = c - lax.slice(c, (jj, 0), (jj + 1, c.shape[1]))
            sqd = jnp.sum(dvec * dvec, axis=1, keepdims=True)
            r_i += jnp.sum(jnp.maximum(_DELTA_R - jnp.sqrt(sqd), 0.0)) - _DELTA_R
        g_i = jnp.sum(jnp.sqrt(cn2))
        lanes = lax.broadcasted_iota(jnp.int32, (1, 128), 1)
        vec = jnp.where(lanes == 0, a_i,
                        jnp.where(lanes == 1, r_i,
                                  jnp.where(lanes == 2, g_i, 0.0)))
        res_ref[0] = vec


def _tc_main(out_r, tgt_r, c_r, lab_r, bn):
    b, d, hw = out_r.shape
    k = c_r.shape[1]
    nb = hw // bn
    return pl.pallas_call(
        _tc_body,
        grid=(b, nb),
        in_specs=[
            pl.BlockSpec((1, d, bn), lambda i, j: (i, 0, j)),
            pl.BlockSpec((1, 1, bn), lambda i, j: (i, 0, j)),
            pl.BlockSpec((1, k, d), lambda i, j: (i, 0, 0)),
            pl.BlockSpec((1, k, 1), lambda i, j: (i, 0, 0)),
        ],
        out_specs=pl.BlockSpec((1, 1, 128), lambda i, j: (i, 0, 0)),
        out_shape=jax.ShapeDtypeStruct((b, 1, 128), jnp.float32),
        scratch_shapes=[
            pltpu.VMEM((k, 128), jnp.float32),
            pltpu.VMEM((k, 128), jnp.float32),
        ],
        compiler_params=pltpu.CompilerParams(
            dimension_semantics=("arbitrary", "arbitrary"),
        ),
    )(out_r, tgt_r, c_r, lab_r)


def _assemble(res, b, k):
    a = res[:, 0, 0]
    r = res[:, 0, 1]
    g = res[:, 0, 2]
    att = jnp.float32(0.0)
    rep = jnp.float32(0.0)
    reg = jnp.float32(0.0)
    for i in range(b):
        att = (att + a[i]) / k
        rep = (rep + r[i]) / (k * (k - 1))
        reg = (reg + g[i]) / k
    loss = _ALPHA * att + _BETA * rep + _GAMMA * reg
    return (loss, att, rep)


def kernel(out, target, centers, batch_size, device):
    b, d, h, w = out.shape
    k = centers.shape[1]
    hw = h * w

    centers = centers.astype(jnp.int32)
    target = target.astype(jnp.int32)
    p = centers[..., 0] * w + centers[..., 1]
    bidx = jnp.arange(b, dtype=jnp.int32)[:, None]
    lab_idx = (bidx * hw + p).reshape(-1)
    d_ar = jnp.arange(d, dtype=jnp.int32)
    emb_idx = ((bidx[:, :, None] * d + d_ar[None, None, :]) * hw
               + p[:, :, None]).reshape(b * k, d)

    p_flat = p.reshape(-1)
    c_g, lab_g = _tc_gather(out.reshape(b, d, hw), target.reshape(b, 1, hw),
                            p_flat // 128, p_flat % 128)
    c_r = c_g.reshape(b, k, d)
    lab_r = lab_g[:, 0, 0].reshape(b, k, 1)

    res = _tc_main(out.reshape(b, d, hw), target.reshape(b, 1, hw),
                   c_r, lab_r, bn=3584)
    return _assemble(res, b, k)


# speedup vs baseline: 7.7220x; 1.1390x over previous
import functools

import jax
import jax.numpy as jnp
from jax import lax
from jax.experimental import pallas as pl
from jax.experimental.pallas import tpu as pltpu
from jax.experimental.pallas import tpu_sc as plsc

_DELTA_A = 0.1
_DELTA_R = 1.0
_ALPHA = 1.0
_BETA = 1.0
_GAMMA = 0.001


def _sc_gather(out_flat, tgt_flat, emb_idx, lab_idx):
    bk, d = emb_idx.shape
    info = plsc.get_sparse_core_info()
    nw = info.num_cores * info.num_subcores
    pairs = bk // nw

    @functools.partial(
        pl.kernel,
        mesh=plsc.VectorSubcoreMesh(core_axis_name="c", subcore_axis_name="s"),
        out_type=[
            jax.ShapeDtypeStruct((bk, d), jnp.float32),
            jax.ShapeDtypeStruct((bk,), jnp.int32),
        ],
        scratch_types=[
            pltpu.VMEM((pairs, d), jnp.int32),
            pltpu.VMEM((pairs, d), jnp.float32),
            pltpu.VMEM((bk,), jnp.int32),
            pltpu.VMEM((bk,), jnp.int32),
            pltpu.SemaphoreType.DMA,
        ],
    )
    def gather_kernel(out_hbm, tgt_hbm, eidx_hbm, lidx_hbm, c_out, lab_out,
                      eidx_v, vals_v, lidx_v, lvals_v, sem):
        wid = lax.axis_index("s") * info.num_cores + lax.axis_index("c")
        base = wid * pairs
        pltpu.sync_copy(eidx_hbm.at[pl.ds(base, pairs)], eidx_v)
        for k in range(pairs):
            pltpu.async_copy(out_hbm.at[eidx_v.at[k]], vals_v.at[k], sem).wait()
        pltpu.sync_copy(vals_v, c_out.at[pl.ds(base, pairs)])

        @pl.when(wid == 0)
        def _():
            pltpu.sync_copy(lidx_hbm, lidx_v)
            pltpu.async_copy(tgt_hbm.at[lidx_v], lvals_v, sem).wait()
            pltpu.sync_copy(lvals_v, lab_out)

    return gather_kernel(out_flat, tgt_flat, emb_idx, lab_idx)


def _make_tc_gather_body(k):
    def body(blk_ref, off_ref, *refs):
        o_refs = refs[:k]
        t_refs = refs[k:2 * k]
        c_ref, cn2_ref, lab_ref = refs[2 * k:]
        i = pl.program_id(0)
        lanes = lax.broadcasted_iota(jnp.int32, (1, 128), 1)
        lanes_k = lax.broadcasted_iota(jnp.int32, (1, k), 1)
        cn2row = jnp.zeros((1, k), jnp.float32)
        labrow = jnp.zeros((1, k), jnp.int32)
        for j in range(k):
            po = off_ref[i * k + j]
            mskf = (lanes == po).astype(jnp.float32)
            col = jnp.sum(o_refs[j][0] * mskf, axis=1, keepdims=True)
            c_ref[0, :, j:j + 1] = col
            cn2row += jnp.sum(col * col) * (lanes_k == j).astype(jnp.float32)
            labv = jnp.sum(t_refs[j][0] * (lanes == po).astype(jnp.int32),
                           axis=1, keepdims=True)
            labrow += labv * (lanes_k == j).astype(jnp.int32)
        cn2_ref[0] = cn2row
        lab_ref[0] = labrow
    return body


def _tc_gather(out_r, tgt_r, blk, off):
    b, d, hw = out_r.shape
    bk = blk.shape[0]
    k = bk // b

    def mk_in(j):
        return pl.BlockSpec((1, d, 128),
                            lambda i, blk, off, j=j: (i, 0, blk[i * k + j]))

    def mk_tin(j):
        return pl.BlockSpec((1, 1, 128),
                            lambda i, blk, off, j=j: (i, 0, blk[i * k + j]))

    grid_spec = pltpu.PrefetchScalarGridSpec(
        num_scalar_prefetch=2,
        grid=(b,),
        in_specs=([mk_in(j) for j in range(k)]
                  + [mk_tin(j) for j in range(k)]),
        out_specs=[
            pl.BlockSpec((1, d, k), lambda i, blk, off: (i, 0, 0)),
            pl.BlockSpec((1, 1, k), lambda i, blk, off: (i, 0, 0)),
            pl.BlockSpec((1, 1, k), lambda i, blk, off: (i, 0, 0)),
        ],
    )
    c2, cn2, lab = pl.pallas_call(
        _make_tc_gather_body(k),
        grid_spec=grid_spec,
        out_shape=[
            jax.ShapeDtypeStruct((b, d, k), jnp.float32),
            jax.ShapeDtypeStruct((b, 1, k), jnp.float32),
            jax.ShapeDtypeStruct((b, 1, k), jnp.int32),
        ],
    )(blk, off, *([out_r] * k), *([tgt_r] * k))
    return c2, cn2, lab


def _tc_body(out_ref, tgt_ref, c_ref, cn2_ref, lab_ref, res_ref,
             attr_ref, cnt_ref):
    j = pl.program_id(1)
    nb = pl.num_programs(1)
    o = out_ref[0]
    t = tgt_ref[0]
    c2 = c_ref[0]
    cn2 = cn2_ref[0]
    lab = lab_ref[0]
    k_centers = c2.shape[1]

    @pl.when(j == 0)
    def _():
        attr_ref[...] = jnp.zeros_like(attr_ref)
        cnt_ref[...] = jnp.zeros_like(cnt_ref)

    g = lax.dot_general(c2, o, (((0,), (0,)), ((), ())),
                        preferred_element_type=jnp.float32,
                        precision=lax.Precision.DEFAULT)
    pn2 = jnp.sum(o * o, axis=0, keepdims=True)
    sq = jnp.maximum(cn2 + pn2 - 2.0 * g, 0.0)
    hinged = jnp.maximum(jnp.sqrt(sq) - _DELTA_A, 0.0)
    m = (t == lab).astype(jnp.float32)
    attr_ref[:, :1] += jnp.sum(hinged * m, axis=1, keepdims=True)
    cnt_ref[:, :1] += jnp.sum(m, axis=1, keepdims=True)

    @pl.when(j == nb - 1)
    def _():
        n = cnt_ref[:, :1]
        denom = jnp.where(n > 1.0, n - 1.0, jnp.maximum(n, 1.0))
        a_i = jnp.sum(attr_ref[:, :1] / denom)
        r_i = jnp.float32(0.0)
        for jj in range(k_centers):
            dvec = c2 - lax.slice(c2, (0, jj), (c2.shape[0], jj + 1))
            sqd = jnp.sum(dvec * dvec, axis=0, keepdims=True)
            r_i += jnp.sum(jnp.maximum(_DELTA_R - jnp.sqrt(sqd), 0.0)) - _DELTA_R
        g_i = jnp.sum(jnp.sqrt(cn2))
        lanes = lax.broadcasted_iota(jnp.int32, (1, 128), 1)
        vec = jnp.where(lanes == 0, a_i,
                        jnp.where(lanes == 1, r_i,
                                  jnp.where(lanes == 2, g_i, 0.0)))
        res_ref[0] = vec


def _tc_main(out_r, tgt_r, c2_r, cn2_r, lab_r, bn):
    b, d, hw = out_r.shape
    k = c2_r.shape[2]
    nb = hw // bn
    return pl.pallas_call(
        _tc_body,
        grid=(b, nb),
        in_specs=[
            pl.BlockSpec((1, d, bn), lambda i, j: (i, 0, j)),
            pl.BlockSpec((1, 1, bn), lambda i, j: (i, 0, j)),
            pl.BlockSpec((1, d, k), lambda i, j: (i, 0, 0)),
            pl.BlockSpec((1, k, 1), lambda i, j: (i, 0, 0)),
            pl.BlockSpec((1, k, 1), lambda i, j: (i, 0, 0)),
        ],
        out_specs=pl.BlockSpec((1, 1, 128), lambda i, j: (i, 0, 0)),
        out_shape=jax.ShapeDtypeStruct((b, 1, 128), jnp.float32),
        scratch_shapes=[
            pltpu.VMEM((k, 128), jnp.float32),
            pltpu.VMEM((k, 128), jnp.float32),
        ],
        compiler_params=pltpu.CompilerParams(
            dimension_semantics=("arbitrary", "arbitrary"),
        ),
    )(out_r, tgt_r, c2_r, cn2_r, lab_r)


def _assemble(res, b, k):
    a = res[:, 0, 0]
    r = res[:, 0, 1]
    g = res[:, 0, 2]
    att = jnp.float32(0.0)
    rep = jnp.float32(0.0)
    reg = jnp.float32(0.0)
    for i in range(b):
        att = (att + a[i]) / k
        rep = (rep + r[i]) / (k * (k - 1))
        reg = (reg + g[i]) / k
    loss = _ALPHA * att + _BETA * rep + _GAMMA * reg
    return (loss, att, rep)


def kernel(out, target, centers, batch_size, device):
    b, d, h, w = out.shape
    k = centers.shape[1]
    hw = h * w

    centers = centers.astype(jnp.int32)
    target = target.astype(jnp.int32)
    p = centers[..., 0] * w + centers[..., 1]
    bidx = jnp.arange(b, dtype=jnp.int32)[:, None]
    lab_idx = (bidx * hw + p).reshape(-1)
    d_ar = jnp.arange(d, dtype=jnp.int32)
    emb_idx = ((bidx[:, :, None] * d + d_ar[None, None, :]) * hw
               + p[:, :, None]).reshape(b * k, d)

    p_flat = p.reshape(-1)
    c2_r, cn2_g, lab_g = _tc_gather(out.reshape(b, d, hw),
                                    target.reshape(b, 1, hw),
                                    p_flat // 128, p_flat % 128)
    cn2_r = cn2_g.reshape(b, k, 1)
    lab_r = lab_g.reshape(b, k, 1)

    res = _tc_main(out.reshape(b, d, hw), target.reshape(b, 1, hw),
                   c2_r, cn2_r, lab_r, bn=3584)
    return _assemble(res, b, k)
